# Initial kernel scaffold; baseline (speedup 1.0000x reference)
import jax
import jax.numpy as jnp
from jax.experimental import pallas as pl


def kernel(x, Wr, br, W1, b1, Wg, bg, W2, b2):
    raise NotImplementedError("write your pallas kernel here")


# speedup vs baseline: 1.2458x; 1.2458x over previous
import functools

import jax
import jax.numpy as jnp
from jax.experimental import pallas as pl
from jax.experimental.pallas import tpu as pltpu

E = 8
TOPK = 2
C = 1024
INNER = 1024
BT = 512


def _moe_body(x_ref, wr_ref, br_ref, w1_ref, b1_ref, wg_ref, bg_ref,
              w2_ref, b2_ref, out_ref):
    e = pl.program_id(1)
    xb = x_ref[...]

    logits = jax.lax.dot_general(
        xb, wr_ref[...], (((1,), (1,)), ((), ())),
        preferred_element_type=jnp.float32) + br_ref[...]
    p = jax.nn.softmax(logits, axis=-1)
    iota_e = jax.lax.broadcasted_iota(jnp.int32, p.shape, 1)
    c1 = jnp.argmax(p, axis=-1)
    p_m = jnp.where(iota_e == c1[:, None], -jnp.inf, p)
    c2 = jnp.argmax(p_m, axis=-1)
    sel = (c1 == e) | (c2 == e)
    pe = jnp.sum(jnp.where(iota_e == e, p, 0.0), axis=-1)
    w = jnp.where(sel, pe, 0.0)

    h1 = jax.lax.dot_general(
        xb, w1_ref[0], (((1,), (1,)), ((), ())),
        preferred_element_type=jnp.float32) + b1_ref[0]
    hg = jax.lax.dot_general(
        xb, wg_ref[0], (((1,), (1,)), ((), ())),
        preferred_element_type=jnp.float32) + bg_ref[0]
    h = (h1 * jax.nn.sigmoid(h1)) * hg
    eo = jax.lax.dot_general(
        h, w2_ref[0], (((1,), (1,)), ((), ())),
        preferred_element_type=jnp.float32) + b2_ref[0]
    contrib = eo * w[:, None]

    @pl.when(e == 0)
    def _():
        out_ref[...] = contrib

    @pl.when(e != 0)
    def _():
        out_ref[...] += contrib


@jax.jit
def kernel(x, Wr, br, W1, b1, Wg, bg, W2, b2):
    B, T, _ = x.shape
    N = B * T
    x2 = x.reshape(N, C)
    nt = N // BT
    out = pl.pallas_call(
        _moe_body,
        grid=(nt, E),
        in_specs=[
            pl.BlockSpec((BT, C), lambda i, e: (i, 0)),
            pl.BlockSpec((E, C), lambda i, e: (0, 0)),
            pl.BlockSpec((1, E), lambda i, e: (0, 0)),
            pl.BlockSpec((1, INNER, C), lambda i, e: (e, 0, 0)),
            pl.BlockSpec((1, 1, INNER), lambda i, e: (e, 0, 0)),
            pl.BlockSpec((1, INNER, C), lambda i, e: (e, 0, 0)),
            pl.BlockSpec((1, 1, INNER), lambda i, e: (e, 0, 0)),
            pl.BlockSpec((1, C, INNER), lambda i, e: (e, 0, 0)),
            pl.BlockSpec((1, 1, C), lambda i, e: (e, 0, 0)),
        ],
        out_specs=pl.BlockSpec((BT, C), lambda i, e: (i, 0)),
        out_shape=jax.ShapeDtypeStruct((N, C), jnp.float32),
        compiler_params=pltpu.CompilerParams(
            dimension_semantics=("arbitrary", "arbitrary")),
    )(x2, Wr, br.reshape(1, E), W1, b1.reshape(E, 1, INNER), Wg,
      bg.reshape(E, 1, INNER), W2, b2.reshape(E, 1, C))
    return out.reshape(B, T, C)


# speedup vs baseline: 1.2533x; 1.0060x over previous
import functools

import jax
import jax.numpy as jnp
from jax import lax
from jax.experimental import pallas as pl
from jax.experimental.pallas import tpu as pltpu
from jax.experimental.pallas import tpu_sc as plsc

E = 8
TOPK = 2
C = 1024
INNER = 1024

BLK = 256
BT_R = 1024

NC, NS = 2, 16
NW = NC * NS
GCH = 64
CCH = 32


def _router_body(x_ref, wr_ref, br_ref, e0_ref, e1_ref, w0_ref, w1_ref):
    xb = x_ref[...]
    logits = lax.dot_general(
        xb, wr_ref[...], (((1,), (1,)), ((), ())),
        preferred_element_type=jnp.float32) + br_ref[...]
    p = jax.nn.softmax(logits, axis=-1)
    iota_e = lax.broadcasted_iota(jnp.int32, p.shape, 1)
    c1 = jnp.argmax(p, axis=-1)
    p1 = jnp.max(p, axis=-1)
    p_m = jnp.where(iota_e == c1[:, None], -jnp.inf, p)
    c2 = jnp.argmax(p_m, axis=-1)
    p2 = jnp.max(p_m, axis=-1)
    e0_ref[...] = c1[:, None].astype(jnp.int32)
    e1_ref[...] = c2[:, None].astype(jnp.int32)
    w0_ref[...] = p1[:, None]
    w1_ref[...] = p2[:, None]


def _sc_gather_body(x_hbm, src_hbm, xs_hbm, idx_v, rows_v, sem):
    wid = lax.axis_index("s") * NC + lax.axis_index("c")
    rows_per_w = xs_hbm.shape[0] // NW
    base = wid * rows_per_w

    def chunk(j, carry):
        off = base + j * GCH
        pltpu.sync_copy(src_hbm.at[pl.ds(off, GCH)], idx_v)
        pltpu.async_copy(x_hbm.at[idx_v], rows_v, sem).wait()
        pltpu.sync_copy(rows_v, xs_hbm.at[pl.ds(off, GCH)])
        return carry

    lax.fori_loop(0, rows_per_w // GCH, chunk, 0)


def _mm_body(be_ref, xs_ref, w1_ref, b1_ref, wg_ref, bg_ref, w2_ref, b2_ref,
             ws_ref, ys_ref):
    xb = xs_ref[...]
    h1 = lax.dot_general(
        xb, w1_ref[0], (((1,), (1,)), ((), ())),
        preferred_element_type=jnp.float32) + b1_ref[0]
    hg = lax.dot_general(
        xb, wg_ref[0], (((1,), (1,)), ((), ())),
        preferred_element_type=jnp.float32) + bg_ref[0]
    h = (h1 * jax.nn.sigmoid(h1)) * hg
    eo = lax.dot_general(
        h, w2_ref[0], (((1,), (1,)), ((), ())),
        preferred_element_type=jnp.float32) + b2_ref[0]
    ys_ref[...] = eo * ws_ref[...]


def _sc_combine_body(ys_hbm, d0_hbm, d1_hbm, out_hbm, i0_v, i1_v, r0_v, r1_v,
                     sem0, sem1):
    wid = lax.axis_index("s") * NC + lax.axis_index("c")
    tok_per_w = out_hbm.shape[0] // NW
    base = wid * tok_per_w

    def chunk(j, carry):
        off = base + j * CCH
        pltpu.sync_copy(d0_hbm.at[pl.ds(off, CCH)], i0_v)
        pltpu.sync_copy(d1_hbm.at[pl.ds(off, CCH)], i1_v)
        cp0 = pltpu.async_copy(ys_hbm.at[i0_v], r0_v, sem0)
        cp1 = pltpu.async_copy(ys_hbm.at[i1_v], r1_v, sem1)
        cp0.wait()
        cp1.wait()

        def row_add(r, c2):
            for k in range(C // 16):
                sl = pl.ds(k * 16, 16)
                r0_v[r, sl] = r0_v[r, sl] + r1_v[r, sl]
            return c2

        lax.fori_loop(0, CCH, row_add, 0)
        pltpu.sync_copy(r0_v, out_hbm.at[pl.ds(off, CCH)])
        return carry

    lax.fori_loop(0, tok_per_w // CCH, chunk, 0)


@jax.jit
def kernel(x, Wr, br, W1, b1, Wg, bg, W2, b2):
    B, T, _ = x.shape
    N = B * T
    P = N * TOPK
    NB = P // BLK + E
    NPAD = NB * BLK
    x2 = x.reshape(N, C)

    e0, e1, w0, w1 = pl.pallas_call(
        _router_body,
        grid=(N // BT_R,),
        in_specs=[
            pl.BlockSpec((BT_R, C), lambda i: (i, 0)),
            pl.BlockSpec((E, C), lambda i: (0, 0)),
            pl.BlockSpec((1, E), lambda i: (0, 0)),
        ],
        out_specs=[
            pl.BlockSpec((BT_R, 1), lambda i: (i, 0)),
            pl.BlockSpec((BT_R, 1), lambda i: (i, 0)),
            pl.BlockSpec((BT_R, 1), lambda i: (i, 0)),
            pl.BlockSpec((BT_R, 1), lambda i: (i, 0)),
        ],
        out_shape=[
            jax.ShapeDtypeStruct((N, 1), jnp.int32),
            jax.ShapeDtypeStruct((N, 1), jnp.int32),
            jax.ShapeDtypeStruct((N, 1), jnp.float32),
            jax.ShapeDtypeStruct((N, 1), jnp.float32),
        ],
    )(x2, Wr, br.reshape(1, E))

    e_flat = jnp.concatenate([e0, e1], axis=1).reshape(P)
    w_flat = jnp.concatenate([w0, w1], axis=1).reshape(P)
    oh = (e_flat[:, None] == jnp.arange(E, dtype=jnp.int32)[None, :])
    csum = jnp.cumsum(oh.astype(jnp.int32), axis=0)
    counts = csum[-1]
    rank = jnp.take_along_axis(csum, e_flat[:, None], axis=1)[:, 0] - 1
    pc = ((counts + BLK - 1) // BLK) * BLK
    ends = jnp.cumsum(pc)
    po = ends - pc
    pos = (po[e_flat] + rank).astype(jnp.int32)
    src = jnp.zeros((NPAD,), jnp.int32).at[pos].set(
        jnp.arange(P, dtype=jnp.int32) // TOPK)
    wsort = jnp.zeros((NPAD,), jnp.float32).at[pos].set(w_flat)
    bstarts = jnp.arange(NB, dtype=jnp.int32) * BLK
    be = jnp.minimum(
        jnp.searchsorted(ends, bstarts, side='right').astype(jnp.int32), E - 1)
    d = pos.reshape(N, TOPK)
    d0, d1 = d[:, 0], d[:, 1]

    mesh = plsc.VectorSubcoreMesh(core_axis_name="c", subcore_axis_name="s", num_cores=NC, num_subcores=NS)
    xs = pl.kernel(
        _sc_gather_body,
        out_type=jax.ShapeDtypeStruct((NPAD, C), jnp.float32),
        mesh=mesh,
        scratch_types=[
            pltpu.VMEM((GCH,), jnp.int32),
            pltpu.VMEM((GCH, C), jnp.float32),
            pltpu.SemaphoreType.DMA,
        ],
    )(x2, src)

    grid_spec = pltpu.PrefetchScalarGridSpec(
        num_scalar_prefetch=1,
        grid=(NB,),
        in_specs=[
            pl.BlockSpec((BLK, C), lambda i, be: (i, 0)),
            pl.BlockSpec((1, INNER, C), lambda i, be: (be[i], 0, 0)),
            pl.BlockSpec((1, 1, INNER), lambda i, be: (be[i], 0, 0)),
            pl.BlockSpec((1, INNER, C), lambda i, be: (be[i], 0, 0)),
            pl.BlockSpec((1, 1, INNER), lambda i, be: (be[i], 0, 0)),
            pl.BlockSpec((1, C, INNER), lambda i, be: (be[i], 0, 0)),
            pl.BlockSpec((1, 1, C), lambda i, be: (be[i], 0, 0)),
            pl.BlockSpec((BLK, 1), lambda i, be: (i, 0)),
        ],
        out_specs=pl.BlockSpec((BLK, C), lambda i, be: (i, 0)),
    )
    ys = pl.pallas_call(
        _mm_body,
        grid_spec=grid_spec,
        out_shape=jax.ShapeDtypeStruct((NPAD, C), jnp.float32),
        compiler_params=pltpu.CompilerParams(
            dimension_semantics=("arbitrary",)),
    )(be, xs, W1, b1.reshape(E, 1, INNER), Wg, bg.reshape(E, 1, INNER),
      W2, b2.reshape(E, 1, C), wsort.reshape(NPAD, 1))

    out = pl.kernel(
        _sc_combine_body,
        out_type=jax.ShapeDtypeStruct((N, C), jnp.float32),
        mesh=plsc.VectorSubcoreMesh(core_axis_name="c", subcore_axis_name="s", num_cores=NC, num_subcores=NS),
        scratch_types=[
            pltpu.VMEM((CCH,), jnp.int32),
            pltpu.VMEM((CCH,), jnp.int32),
            pltpu.VMEM((CCH, C), jnp.float32),
            pltpu.VMEM((CCH, C), jnp.float32),
            pltpu.SemaphoreType.DMA,
            pltpu.SemaphoreType.DMA,
        ],
    )(ys, d0, d1)

    return out.reshape(B, T, C)
